# Initial kernel scaffold; baseline (speedup 1.0000x reference)
import jax
import jax.numpy as jnp
from jax.experimental import pallas as pl


def kernel(in_feat, edge_index, W1_self, W1_neigh, b1, W2_self, W2_neigh, b2):
    raise NotImplementedError("write your pallas kernel here")


# speedup vs baseline: 3.3691x; 3.3691x over previous
import functools

import jax
import jax.numpy as jnp
from jax import lax
from jax.experimental import pallas as pl
from jax.experimental.pallas import tpu as pltpu
from jax.experimental.pallas import tpu_sc as plsc

N = 10000
E = 320000
D = 128
NPAD = 10240
NC = 2
NS = 16
NW = NC * NS
B = 128
KC = 8
K = KC * (-(-E // (NW * B * KC)))
EPAD = NW * K * B
ROWS_PT = NPAD // NS
R = 1024


def _zero_block(rows_v):
    def zrow_i(i, c):
        def zrow_j(j, c2):
            rows_v[i, pl.ds(j * 16, 16)] = jnp.zeros((16,), jnp.float32)
            return c2
        return lax.fori_loop(0, D // 16, zrow_j, c)
    lax.fori_loop(0, B, zrow_i, 0)


def _sc_aggregate(y, srcp, dstp):
    out_types = (jax.ShapeDtypeStruct((NC, NPAD, D), jnp.float32),)
    scratch = [
        pltpu.VMEM_SHARED((NPAD, D), jnp.float32),
        pltpu.VMEM((KC, B), jnp.int32),
        pltpu.VMEM((KC, B), jnp.int32),
        pltpu.VMEM((B, D), jnp.float32),
        pltpu.SemaphoreType.DMA,
    ]
    mesh = plsc.VectorSubcoreMesh(core_axis_name="c", subcore_axis_name="s")

    def body(y_hbm, src_hbm, dst_hbm, part_hbm, acc_sh,
             src_v, dst_v, rows_v, sem):
        cid = lax.axis_index("c")
        sid = lax.axis_index("s")
        wid = sid * NC + cid
        base = sid * ROWS_PT

        _zero_block(rows_v)
        for t in range(ROWS_PT // B):
            pltpu.sync_copy(rows_v, acc_sh.at[pl.ds(base + t * B, B)])
        plsc.subcore_barrier()

        def outer(o, c):
            pltpu.sync_copy(src_hbm.at[wid, pl.ds(o * KC, KC)], src_v)
            pltpu.sync_copy(dst_hbm.at[wid, pl.ds(o * KC, KC)], dst_v)

            def edge_batch(j, c2):
                pltpu.async_copy(y_hbm.at[src_v.at[j]], rows_v, sem).wait()
                pltpu.sync_copy(rows_v, acc_sh.at[dst_v.at[j]], add=True)
                return c2
            return lax.fori_loop(0, KC, edge_batch, c)
        lax.fori_loop(0, K // KC, outer, 0)

        plsc.subcore_barrier()
        pltpu.sync_copy(acc_sh.at[pl.ds(base, ROWS_PT)],
                        part_hbm.at[cid, pl.ds(base, ROWS_PT)])

    fn = pl.kernel(body, out_type=out_types, mesh=mesh,
                   scratch_types=scratch)
    return fn(y, srcp, dstp)[0]


def _sc_degree(dstp):
    out_types = (jax.ShapeDtypeStruct((NC, NPAD, D), jnp.float32),)
    scratch = [
        pltpu.VMEM_SHARED((NPAD, D), jnp.float32),
        pltpu.VMEM((KC, B), jnp.int32),
        pltpu.VMEM((B, D), jnp.float32),
    ]
    mesh = plsc.VectorSubcoreMesh(core_axis_name="c", subcore_axis_name="s")

    def body(dst_hbm, degp_hbm, deg_sh, dst_v, ones_v):
        cid = lax.axis_index("c")
        sid = lax.axis_index("s")
        wid = sid * NC + cid
        base = sid * ROWS_PT

        _zero_block(ones_v)
        for t in range(ROWS_PT // B):
            pltpu.sync_copy(ones_v, deg_sh.at[pl.ds(base + t * B, B)])

        def one_i(i, c):
            def one_j(j, c2):
                ones_v[i, pl.ds(j * 16, 16)] = jnp.ones((16,), jnp.float32)
                return c2
            return lax.fori_loop(0, D // 16, one_j, c)
        lax.fori_loop(0, B, one_i, 0)
        plsc.subcore_barrier()

        def outer(o, c):
            pltpu.sync_copy(dst_hbm.at[wid, pl.ds(o * KC, KC)], dst_v)

            def edge_batch(j, c2):
                pltpu.sync_copy(ones_v, deg_sh.at[dst_v.at[j]], add=True)
                return c2
            return lax.fori_loop(0, KC, edge_batch, c)
        lax.fori_loop(0, K // KC, outer, 0)

        plsc.subcore_barrier()
        pltpu.sync_copy(deg_sh.at[pl.ds(base, ROWS_PT)],
                        degp_hbm.at[cid, pl.ds(base, ROWS_PT)])

    fn = pl.kernel(body, out_type=out_types, mesh=mesh,
                   scratch_types=scratch)
    return fn(dstp)[0]


def _tc_pre(x, ws, wn, b):
    def body(x_ref, ws_ref, wn_ref, b_ref, s_ref, y_ref):
        xb = x_ref[...]
        s_ref[...] = jnp.dot(xb, ws_ref[...],
                             preferred_element_type=jnp.float32) + b_ref[...]
        y_ref[...] = jnp.dot(xb, wn_ref[...],
                             preferred_element_type=jnp.float32)
    return pl.pallas_call(
        body,
        grid=(NPAD // R,),
        in_specs=[
            pl.BlockSpec((R, D), lambda i: (i, 0)),
            pl.BlockSpec((D, D), lambda i: (0, 0)),
            pl.BlockSpec((D, D), lambda i: (0, 0)),
            pl.BlockSpec((1, D), lambda i: (0, 0)),
        ],
        out_specs=[pl.BlockSpec((R, D), lambda i: (i, 0)),
                   pl.BlockSpec((R, D), lambda i: (i, 0))],
        out_shape=[jax.ShapeDtypeStruct((NPAD, D), jnp.float32)] * 2,
    )(x, ws, wn, b)


def _tc_mid(s1, parts, degp, ws, wn, b):
    def body(s1_ref, p_ref, dg_ref, ws_ref, wn_ref, b_ref, s2_ref, y2_ref):
        agg = p_ref[0] + p_ref[1]
        deg = dg_ref[0, :, 0:1] + dg_ref[1, :, 0:1]
        rdeg = 1.0 / jnp.maximum(deg, 1.0)
        h1 = jnp.maximum(s1_ref[...] + agg * rdeg, 0.0)
        s2_ref[...] = jnp.dot(h1, ws_ref[...],
                              preferred_element_type=jnp.float32) + b_ref[...]
        y2_ref[...] = jnp.dot(h1, wn_ref[...],
                              preferred_element_type=jnp.float32)
    return pl.pallas_call(
        body,
        grid=(NPAD // R,),
        in_specs=[
            pl.BlockSpec((R, D), lambda i: (i, 0)),
            pl.BlockSpec((NC, R, D), lambda i: (0, i, 0)),
            pl.BlockSpec((NC, R, D), lambda i: (0, i, 0)),
            pl.BlockSpec((D, D), lambda i: (0, 0)),
            pl.BlockSpec((D, D), lambda i: (0, 0)),
            pl.BlockSpec((1, D), lambda i: (0, 0)),
        ],
        out_specs=[pl.BlockSpec((R, D), lambda i: (i, 0)),
                   pl.BlockSpec((R, D), lambda i: (i, 0))],
        out_shape=[jax.ShapeDtypeStruct((NPAD, D), jnp.float32)] * 2,
    )(s1, parts, degp, ws, wn, b)


def _tc_post(s2, parts, degp):
    def body(s2_ref, p_ref, dg_ref, o_ref):
        agg = p_ref[0] + p_ref[1]
        deg = dg_ref[0, :, 0:1] + dg_ref[1, :, 0:1]
        o_ref[...] = s2_ref[...] + agg * (1.0 / jnp.maximum(deg, 1.0))
    return pl.pallas_call(
        body,
        grid=(NPAD // R,),
        in_specs=[
            pl.BlockSpec((R, D), lambda i: (i, 0)),
            pl.BlockSpec((NC, R, D), lambda i: (0, i, 0)),
            pl.BlockSpec((NC, R, D), lambda i: (0, i, 0)),
        ],
        out_specs=pl.BlockSpec((R, D), lambda i: (i, 0)),
        out_shape=jax.ShapeDtypeStruct((NPAD, D), jnp.float32),
    )(s2, parts, degp)


def kernel(in_feat, edge_index, W1_self, W1_neigh, b1, W2_self, W2_neigh, b2):
    src = edge_index[0].astype(jnp.int32)
    dst = edge_index[1].astype(jnp.int32)
    pad = EPAD - E
    srcp = jnp.concatenate([src, jnp.zeros((pad,), jnp.int32)]).reshape(NW, K, B)
    dstp = jnp.concatenate([dst, jnp.full((pad,), N, jnp.int32)]).reshape(NW, K, B)
    x = jnp.pad(in_feat, ((0, NPAD - N), (0, 0)))

    degp = _sc_degree(dstp)
    s1, y1 = _tc_pre(x, W1_self, W1_neigh, b1.reshape(1, D))
    parts1 = _sc_aggregate(y1, srcp, dstp)
    s2, y2 = _tc_mid(s1, parts1, degp, W2_self, W2_neigh, b2.reshape(1, D))
    parts2 = _sc_aggregate(y2, srcp, dstp)
    out = _tc_post(s2, parts2, degp)
    return out[:N]


# speedup vs baseline: 3.6680x; 1.0887x over previous
import functools

import jax
import jax.numpy as jnp
from jax import lax
from jax.experimental import pallas as pl
from jax.experimental.pallas import tpu as pltpu
from jax.experimental.pallas import tpu_sc as plsc

N = 10000
E = 320000
D = 128
NPAD = 10240
NC = 2
NS = 16
NW = NC * NS
B = 128
KC = 8
KCI = 16
K = KCI * (-(-E // (NW * B * KCI)))
EPAD = NW * K * B
ROWS_PT = NPAD // NS
R = 1024


def _zero_block(rows_v):
    def zrow_i(i, c):
        def zrow_j(j, c2):
            rows_v[i, pl.ds(j * 16, 16)] = jnp.zeros((16,), jnp.float32)
            return c2
        return lax.fori_loop(0, D // 16, zrow_j, c)
    lax.fori_loop(0, B, zrow_i, 0)


def _sc_aggregate(y, srcp, dstp):
    out_types = (jax.ShapeDtypeStruct((NC, NPAD, D), jnp.float32),)
    scratch = [
        pltpu.VMEM_SHARED((NPAD, D), jnp.float32),
        pltpu.VMEM((KCI, B), jnp.int32),
        pltpu.VMEM((KCI, B), jnp.int32),
        pltpu.VMEM((2, B, D), jnp.float32),
        pltpu.SemaphoreType.DMA,
        pltpu.SemaphoreType.DMA,
    ]
    mesh = plsc.VectorSubcoreMesh(core_axis_name="c", subcore_axis_name="s")

    def body(y_hbm, src_hbm, dst_hbm, part_hbm, acc_sh,
             src_v, dst_v, rows_v, sem0, sem1):
        cid = lax.axis_index("c")
        sid = lax.axis_index("s")
        wid = sid * NC + cid
        base = sid * ROWS_PT
        sems = (sem0, sem1)

        _zero_block(rows_v.at[0])
        for t in range(ROWS_PT // B):
            pltpu.sync_copy(rows_v.at[0], acc_sh.at[pl.ds(base + t * B, B)])
        plsc.subcore_barrier()

        def outer(o, c):
            pltpu.sync_copy(src_hbm.at[wid, pl.ds(o * KCI, KCI)], src_v)
            pltpu.sync_copy(dst_hbm.at[wid, pl.ds(o * KCI, KCI)], dst_v)
            cps = [None, None]
            cps[0] = pltpu.async_copy(y_hbm.at[src_v.at[0]], rows_v.at[0],
                                      sems[0])
            for j in range(KCI):
                p = j % 2
                cps[p].wait()
                if j + 1 < KCI:
                    cps[1 - p] = pltpu.async_copy(
                        y_hbm.at[src_v.at[j + 1]], rows_v.at[1 - p],
                        sems[1 - p])
                pltpu.sync_copy(rows_v.at[p], acc_sh.at[dst_v.at[j]],
                                add=True)
            return c
        lax.fori_loop(0, K // KCI, outer, 0)

        plsc.subcore_barrier()
        pltpu.sync_copy(acc_sh.at[pl.ds(base, ROWS_PT)],
                        part_hbm.at[cid, pl.ds(base, ROWS_PT)])

    fn = pl.kernel(body, out_type=out_types, mesh=mesh,
                   scratch_types=scratch)
    return fn(y, srcp, dstp)[0]


def _sc_degree(dstp):
    out_types = (jax.ShapeDtypeStruct((NC, NPAD, D), jnp.float32),)
    scratch = [
        pltpu.VMEM_SHARED((NPAD, D), jnp.float32),
        pltpu.VMEM((KC, B), jnp.int32),
        pltpu.VMEM((B, D), jnp.float32),
    ]
    mesh = plsc.VectorSubcoreMesh(core_axis_name="c", subcore_axis_name="s")

    def body(dst_hbm, degp_hbm, deg_sh, dst_v, ones_v):
        cid = lax.axis_index("c")
        sid = lax.axis_index("s")
        wid = sid * NC + cid
        base = sid * ROWS_PT

        _zero_block(ones_v)
        for t in range(ROWS_PT // B):
            pltpu.sync_copy(ones_v, deg_sh.at[pl.ds(base + t * B, B)])

        def one_i(i, c):
            def one_j(j, c2):
                ones_v[i, pl.ds(j * 16, 16)] = jnp.ones((16,), jnp.float32)
                return c2
            return lax.fori_loop(0, D // 16, one_j, c)
        lax.fori_loop(0, B, one_i, 0)
        plsc.subcore_barrier()

        def outer(o, c):
            pltpu.sync_copy(dst_hbm.at[wid, pl.ds(o * KC, KC)], dst_v)

            def edge_batch(j, c2):
                pltpu.sync_copy(ones_v, deg_sh.at[dst_v.at[j]], add=True)
                return c2
            return lax.fori_loop(0, KC, edge_batch, c)
        lax.fori_loop(0, K // KC, outer, 0)

        plsc.subcore_barrier()
        pltpu.sync_copy(deg_sh.at[pl.ds(base, ROWS_PT)],
                        degp_hbm.at[cid, pl.ds(base, ROWS_PT)])

    fn = pl.kernel(body, out_type=out_types, mesh=mesh,
                   scratch_types=scratch)
    return fn(dstp)[0]


def _tc_pre(x, ws, wn, b):
    def body(x_ref, ws_ref, wn_ref, b_ref, s_ref, y_ref):
        xb = x_ref[...]
        s_ref[...] = jnp.dot(xb, ws_ref[...],
                             preferred_element_type=jnp.float32) + b_ref[...]
        y_ref[...] = jnp.dot(xb, wn_ref[...],
                             preferred_element_type=jnp.float32)
    return pl.pallas_call(
        body,
        grid=(NPAD // R,),
        in_specs=[
            pl.BlockSpec((R, D), lambda i: (i, 0)),
            pl.BlockSpec((D, D), lambda i: (0, 0)),
            pl.BlockSpec((D, D), lambda i: (0, 0)),
            pl.BlockSpec((1, D), lambda i: (0, 0)),
        ],
        out_specs=[pl.BlockSpec((R, D), lambda i: (i, 0)),
                   pl.BlockSpec((R, D), lambda i: (i, 0))],
        out_shape=[jax.ShapeDtypeStruct((NPAD, D), jnp.float32)] * 2,
    )(x, ws, wn, b)


def _tc_mid(s1, parts, degp, ws, wn, b):
    def body(s1_ref, p_ref, dg_ref, ws_ref, wn_ref, b_ref, s2_ref, y2_ref):
        agg = p_ref[0] + p_ref[1]
        deg = dg_ref[0, :, 0:1] + dg_ref[1, :, 0:1]
        rdeg = 1.0 / jnp.maximum(deg, 1.0)
        h1 = jnp.maximum(s1_ref[...] + agg * rdeg, 0.0)
        s2_ref[...] = jnp.dot(h1, ws_ref[...],
                              preferred_element_type=jnp.float32) + b_ref[...]
        y2_ref[...] = jnp.dot(h1, wn_ref[...],
                              preferred_element_type=jnp.float32)
    return pl.pallas_call(
        body,
        grid=(NPAD // R,),
        in_specs=[
            pl.BlockSpec((R, D), lambda i: (i, 0)),
            pl.BlockSpec((NC, R, D), lambda i: (0, i, 0)),
            pl.BlockSpec((NC, R, D), lambda i: (0, i, 0)),
            pl.BlockSpec((D, D), lambda i: (0, 0)),
            pl.BlockSpec((D, D), lambda i: (0, 0)),
            pl.BlockSpec((1, D), lambda i: (0, 0)),
        ],
        out_specs=[pl.BlockSpec((R, D), lambda i: (i, 0)),
                   pl.BlockSpec((R, D), lambda i: (i, 0))],
        out_shape=[jax.ShapeDtypeStruct((NPAD, D), jnp.float32)] * 2,
    )(s1, parts, degp, ws, wn, b)


def _tc_post(s2, parts, degp):
    def body(s2_ref, p_ref, dg_ref, o_ref):
        agg = p_ref[0] + p_ref[1]
        deg = dg_ref[0, :, 0:1] + dg_ref[1, :, 0:1]
        o_ref[...] = s2_ref[...] + agg * (1.0 / jnp.maximum(deg, 1.0))
    return pl.pallas_call(
        body,
        grid=(NPAD // R,),
        in_specs=[
            pl.BlockSpec((R, D), lambda i: (i, 0)),
            pl.BlockSpec((NC, R, D), lambda i: (0, i, 0)),
            pl.BlockSpec((NC, R, D), lambda i: (0, i, 0)),
        ],
        out_specs=pl.BlockSpec((R, D), lambda i: (i, 0)),
        out_shape=jax.ShapeDtypeStruct((NPAD, D), jnp.float32),
    )(s2, parts, degp)


def kernel(in_feat, edge_index, W1_self, W1_neigh, b1, W2_self, W2_neigh, b2):
    src = edge_index[0].astype(jnp.int32)
    dst = edge_index[1].astype(jnp.int32)
    pad = EPAD - E
    srcp = jnp.concatenate([src, jnp.zeros((pad,), jnp.int32)]).reshape(NW, K, B)
    dstp = jnp.concatenate([dst, jnp.full((pad,), N, jnp.int32)]).reshape(NW, K, B)
    x = jnp.pad(in_feat, ((0, NPAD - N), (0, 0)))

    degp = _sc_degree(dstp)
    s1, y1 = _tc_pre(x, W1_self, W1_neigh, b1.reshape(1, D))
    parts1 = _sc_aggregate(y1, srcp, dstp)
    s2, y2 = _tc_mid(s1, parts1, degp, W2_self, W2_neigh, b2.reshape(1, D))
    parts2 = _sc_aggregate(y2, srcp, dstp)
    out = _tc_post(s2, parts2, degp)
    return out[:N]
